# baseline (speedup 1.0000x reference)
import functools

import jax
import jax.numpy as jnp
import numpy as np
from jax import lax
from jax.experimental import pallas as pl
from jax.experimental.pallas import tpu as pltpu
from jax.experimental.pallas import tpu_sc as plsc

NUM_INTERVALS = 128
MAX_TIME = 1.0
D_FEAT = 128
N_ROWS = 131072

NC = 2
NS = 16
L = 16
NV = NUM_INTERVALS // L

CHUNK = 128
N_SC = 73728
NCHUNKS = N_SC // (NC * NS) // CHUNK

TC_ROWS = N_ROWS - N_SC
TC_BLOCK = 512
TC_OFF_BLOCKS = N_SC // TC_BLOCK


def _sc_body(t_hbm, z_hbm, ind_hbm, dt_hbm, dtind_hbm, tauind_hbm,
             taunext_hbm, vin, tin, dtbuf, taubuf, indbuf, dtindbuf,
             tauindbuf, taunextbuf, insem, outsem):
    c = lax.axis_index("c")
    s = lax.axis_index("s")
    wid = c * NS + s
    rows_per = N_SC // (NC * NS)
    base = wid * rows_per
    lane = lax.iota(jnp.int32, L)

    def in_copies(ci, p):
        row0 = base + ci * CHUNK
        return (
            pltpu.make_async_copy(
                z_hbm.at[pl.ds(row0, CHUNK), pl.ds(D_FEAT, NUM_INTERVALS)],
                vin.at[p], insem.at[p]),
            pltpu.make_async_copy(t_hbm.at[pl.ds(row0, CHUNK)], tin.at[p],
                                  insem.at[p]),
        )

    def out_copies(ci, p):
        row0 = base + ci * CHUNK
        dst = pl.ds(row0, CHUNK)
        return (
            pltpu.make_async_copy(dtbuf.at[p], dt_hbm.at[dst], outsem.at[p]),
            pltpu.make_async_copy(indbuf.at[p], ind_hbm.at[dst], outsem.at[p]),
            pltpu.make_async_copy(dtindbuf.at[p], dtind_hbm.at[dst],
                                  outsem.at[p]),
            pltpu.make_async_copy(tauindbuf.at[p], tauind_hbm.at[dst],
                                  outsem.at[p]),
            pltpu.make_async_copy(taunextbuf.at[p], taunext_hbm.at[dst],
                                  outsem.at[p]),
        )

    for cp in in_copies(0, 0):
        cp.start()
    for cp in in_copies(1, 1):
        cp.start()

    def chunk_body(ci, carry):
        p = jnp.bitwise_and(ci, 1)
        for cp in in_copies(ci, p):
            cp.wait()

        @pl.when(ci >= 2)
        def _():
            for cp in out_copies(ci - 2, p):
                cp.wait()

        def group_body(g, gcarry):
            ind_acc = jnp.zeros((L,), jnp.int32)
            t16 = tin[p, pl.ds(g * L, L)]
            for j in range(L):
                r = g * L + j
                tj = t16[j]
                e = []
                cume = []
                for i in range(NV):
                    ei = jnp.exp(vin[p, r, pl.ds(i * L, L)])
                    e.append(ei)
                    cume.append(plsc.cumsum(ei))
                prefix = []
                tot = np.float32(0.0)
                for i in range(NV):
                    prefix.append(tot)
                    tot = tot + cume[i][L - 1]
                inv = 1.0 / jnp.broadcast_to(tot, (L,))
                cnt = jnp.zeros((L,), jnp.int32)
                for i in range(NV):
                    dtbuf[p, r, pl.ds(i * L, L)] = e[i] * inv
                    taui = (cume[i] + prefix[i]) * inv
                    taubuf[j, pl.ds(i * L, L)] = taui
                    m = taui < tj
                    if i == NV - 1:
                        m = m & (lane < L - 1)
                    cnt = cnt + m.astype(jnp.int32)
                indj = jnp.sum(cnt)
                ind_acc = ind_acc + jnp.where(lane == j, indj, 0)
            rows16 = g * L + lane
            p16 = jnp.broadcast_to(p, (L,))
            dtind16 = plsc.load_gather(dtbuf, [p16, rows16, ind_acc])
            taunext16 = plsc.load_gather(taubuf, [lane, ind_acc])
            indbuf[p, pl.ds(g * L, L)] = ind_acc
            dtindbuf[p, pl.ds(g * L, L)] = dtind16
            taunextbuf[p, pl.ds(g * L, L)] = taunext16
            tauindbuf[p, pl.ds(g * L, L)] = taunext16 - dtind16
            return gcarry

        lax.fori_loop(0, CHUNK // L, group_body, 0)

        for cp in out_copies(ci, p):
            cp.start()

        @pl.when(ci + 2 < NCHUNKS)
        def _():
            for cp in in_copies(ci + 2, p):
                cp.start()

        return carry

    lax.fori_loop(0, NCHUNKS, chunk_body, 0)

    for cp in out_copies(NCHUNKS - 2, 0):
        cp.wait()
    for cp in out_copies(NCHUNKS - 1, 1):
        cp.wait()


def _tc_body(t_ref, z_ref, ind_ref, dt_ref, dtind_ref, tauind_ref,
             taunext_ref):
    z = z_ref[...]
    v = z[:, D_FEAT:]
    m = jnp.max(v, axis=1, keepdims=True)
    e = jnp.exp(v - m)
    s = jnp.sum(e, axis=1, keepdims=True)
    dt = e / s
    dt_ref[...] = dt

    lanes = lax.broadcasted_iota(jnp.int32, dt.shape, 1)
    tau = dt
    k = 1
    while k < NUM_INTERVALS:
        tau = tau + jnp.where(lanes >= k, pltpu.roll(tau, k, 1), 0.0)
        k *= 2

    t = t_ref[...]
    below = (tau < t) & (lanes < NUM_INTERVALS - 1)
    ind = jnp.sum(below.astype(jnp.int32), axis=1, keepdims=True)
    ind_ref[...] = ind

    onehot = (lanes == ind).astype(jnp.float32)
    tau_next = jnp.sum(tau * onehot, axis=1, keepdims=True)
    dt_ind = jnp.sum(dt * onehot, axis=1, keepdims=True)
    taunext_ref[...] = tau_next
    dtind_ref[...] = dt_ind
    tauind_ref[...] = tau_next - dt_ind


def _tc_tail(t2d, z):
    r = TC_BLOCK
    grid = (TC_ROWS // r,)
    out_shapes = (
        jax.ShapeDtypeStruct((TC_ROWS, 1), jnp.int32),
        jax.ShapeDtypeStruct((TC_ROWS, NUM_INTERVALS), jnp.float32),
        jax.ShapeDtypeStruct((TC_ROWS, 1), jnp.float32),
        jax.ShapeDtypeStruct((TC_ROWS, 1), jnp.float32),
        jax.ShapeDtypeStruct((TC_ROWS, 1), jnp.float32),
    )
    col = lambda i: (i, 0)
    off = lambda i: (i + TC_OFF_BLOCKS, 0)
    out_specs = (
        pl.BlockSpec((r, 1), col),
        pl.BlockSpec((r, NUM_INTERVALS), col),
        pl.BlockSpec((r, 1), col),
        pl.BlockSpec((r, 1), col),
        pl.BlockSpec((r, 1), col),
    )
    in_specs = [
        pl.BlockSpec((r, 1), off),
        pl.BlockSpec((r, D_FEAT + NUM_INTERVALS), off),
    ]
    return pl.pallas_call(
        _tc_body,
        grid=grid,
        in_specs=in_specs,
        out_specs=out_specs,
        out_shape=out_shapes,
    )(t2d, z)


@jax.jit
def kernel(t, z):
    n = t.shape[0]
    mesh = plsc.VectorSubcoreMesh(core_axis_name="c", subcore_axis_name="s")
    out_type = (
        jax.ShapeDtypeStruct((n,), jnp.int32),
        jax.ShapeDtypeStruct((n, NUM_INTERVALS), jnp.float32),
        jax.ShapeDtypeStruct((n,), jnp.float32),
        jax.ShapeDtypeStruct((n,), jnp.float32),
        jax.ShapeDtypeStruct((n,), jnp.float32),
    )
    scratch = [
        pltpu.VMEM((2, CHUNK, NUM_INTERVALS), jnp.float32),
        pltpu.VMEM((2, CHUNK), jnp.float32),
        pltpu.VMEM((2, CHUNK, NUM_INTERVALS), jnp.float32),
        pltpu.VMEM((L, NUM_INTERVALS), jnp.float32),
        pltpu.VMEM((2, CHUNK), jnp.int32),
        pltpu.VMEM((2, CHUNK), jnp.float32),
        pltpu.VMEM((2, CHUNK), jnp.float32),
        pltpu.VMEM((2, CHUNK), jnp.float32),
        pltpu.SemaphoreType.DMA((2,)),
        pltpu.SemaphoreType.DMA((2,)),
    ]
    ind_f, dt_f, dtind_f, tauind_f, taunext_f = pl.kernel(
        _sc_body,
        out_type=out_type,
        mesh=mesh,
        scratch_types=scratch,
        compiler_params=pltpu.CompilerParams(needs_layout_passes=False),
    )(t, z)

    ind_t, dt_t, dtind_t, tauind_t, taunext_t = _tc_tail(t.reshape(n, 1), z)

    upd = lambda full, tail: lax.dynamic_update_slice(
        full, tail, (N_SC,) + (0,) * (tail.ndim - 1))
    ind = upd(ind_f, ind_t.reshape(TC_ROWS).astype(jnp.int32))
    dt = upd(dt_f, dt_t)
    dt_ind = upd(dtind_f, dtind_t.reshape(TC_ROWS))
    tau_ind = upd(tauind_f, tauind_t.reshape(TC_ROWS))
    tau_next = upd(taunext_f, taunext_t.reshape(TC_ROWS))
    z0 = z[:, :D_FEAT]
    return (ind, dt, dt_ind, tau_ind, tau_next, z0)
